# baseline (speedup 1.0000x reference)
import jax
import jax.numpy as jnp
from jax import lax
from jax.experimental import pallas as pl

B, F, C = 4096, 100, 100
FA, CA = 104, 128
BB = 256
NEG = -1


def _onehot_body(idx_ref, out_ref):
    ones = jnp.ones((1, CA), jnp.float32)
    iota = lax.broadcasted_iota(jnp.int32, (FA, CA), 1).astype(jnp.float32)
    for b in range(BB):
        x = idx_ref[b : b + 1, :]
        splat = lax.dot_general(
            x, ones, (((0,), (0,)), ((), ())),
            preferred_element_type=jnp.float32,
        )
        out_ref[b] = (splat == iota).astype(jnp.int32)


def kernel(tensor):
    idx = tensor.reshape(B, F)
    idxp = jnp.pad(idx, ((0, 0), (0, FA - F)), constant_values=NEG)
    idxf = idxp.astype(jnp.float32)
    H = B // 2
    call = pl.pallas_call(
        _onehot_body,
        grid=(H // BB,),
        in_specs=[pl.BlockSpec((BB, FA), lambda i: (i, 0))],
        out_specs=pl.BlockSpec((BB, FA, CA), lambda i: (i, 0, 0)),
        out_shape=jax.ShapeDtypeStruct((H, FA, CA), jnp.int32),
    )
    halves = [call(idxf[h * H : (h + 1) * H])[:, :F, :C] for h in range(2)]
    return jnp.concatenate(halves, axis=0)


# speedup vs baseline: 1.6333x; 1.6333x over previous
import jax
import jax.numpy as jnp
from jax import lax
from jax.experimental import pallas as pl

B, F, C = 4096, 100, 100
FA, CA = 104, 128
BB = 256
NEG = -1


def _onehot_body(idx_ref, out_ref):
    ones = jnp.ones((1, CA), jnp.float32)
    iota = lax.broadcasted_iota(jnp.int32, (FA, CA), 1).astype(jnp.float32)
    for b in range(BB):
        x = idx_ref[b : b + 1, :]
        splat = lax.dot_general(
            x, ones, (((0,), (0,)), ((), ())),
            preferred_element_type=jnp.float32,
        )
        out_ref[b] = (splat == iota).astype(jnp.int32)


def kernel(tensor):
    idx = tensor.reshape(B, F)
    idxp = jnp.pad(idx, ((0, 0), (0, FA - F)), constant_values=NEG)
    idxf = idxp.astype(jnp.float32)
    big = pl.pallas_call(
        _onehot_body,
        grid=(B // BB,),
        in_specs=[pl.BlockSpec((BB, FA), lambda i: (i, 0))],
        out_specs=pl.BlockSpec((BB, FA, CA), lambda i: (i, 0, 0)),
        out_shape=jax.ShapeDtypeStruct((B, FA, CA), jnp.int32),
    )(idxf)
    return big[:, :F, :C]


# speedup vs baseline: 1.6421x; 1.0054x over previous
import jax
import jax.numpy as jnp
from jax import lax
from jax.experimental import pallas as pl

B, F, C = 4096, 100, 100
FA, CA = 104, 128
BB = 512
NEG = -1


def _onehot_body(idx_ref, out_ref):
    ones = jnp.ones((1, CA), jnp.float32)
    iota = lax.broadcasted_iota(jnp.int32, (FA, CA), 1).astype(jnp.float32)
    for b in range(BB):
        x = idx_ref[b : b + 1, :]
        splat = lax.dot_general(
            x, ones, (((0,), (0,)), ((), ())),
            preferred_element_type=jnp.float32,
        )
        out_ref[b] = (splat == iota).astype(jnp.int32)


def kernel(tensor):
    idx = tensor.reshape(B, F)
    idxp = jnp.pad(idx, ((0, 0), (0, FA - F)), constant_values=NEG)
    idxf = idxp.astype(jnp.float32)
    big = pl.pallas_call(
        _onehot_body,
        grid=(B // BB,),
        in_specs=[pl.BlockSpec((BB, FA), lambda i: (i, 0))],
        out_specs=pl.BlockSpec((BB, FA, CA), lambda i: (i, 0, 0)),
        out_shape=jax.ShapeDtypeStruct((B, FA, CA), jnp.int32),
    )(idxf)
    return big[:, :F, :C]
